# baseline (speedup 1.0000x reference)
import functools

import jax
import jax.numpy as jnp
from jax import lax
from jax.experimental import pallas as pl
from jax.experimental.pallas import tpu as pltpu
from jax.experimental.pallas import tpu_sc as plsc

_N_GENES = 1024
_N_SEG = 100
_WTAB = _N_GENES * _N_SEG
_HSIZE = 102912
_DUMMY = _WTAB
_N_FRAG = 1_000_000
_N_MOT = 2_000_000

_MOT_PER = 62496
_MOT_CH = 2048
_MOT_FULL = 30
_MOT_REM0 = _MOT_PER - _MOT_FULL * _MOT_CH
_MOT_REM1 = (_N_MOT - 31 * _MOT_PER) - _MOT_FULL * _MOT_CH
_MSLICE = _WTAB // 16

_FR_PER = 31248
_FR_CH = 1024
_FR_FULL = 30
_FR_REM0 = _FR_PER - _FR_FULL * _FR_CH
_FR_REM1 = (_N_FRAG - 31 * _FR_PER) - _FR_FULL * _FR_CH

_mesh = plsc.VectorSubcoreMesh(core_axis_name="c", subcore_axis_name="s")


@functools.partial(
    pl.kernel,
    mesh=_mesh,
    out_type=jax.ShapeDtypeStruct((32, _WTAB), jnp.int32),
    compiler_params=pltpu.CompilerParams(needs_layout_passes=False),
    scratch_types=[
        pltpu.VMEM((_MOT_CH,), jnp.int32),
        pltpu.VMEM((_MOT_CH,), jnp.int32),
        pltpu.VMEM((_MOT_CH,), jnp.int32),
        pltpu.VMEM((_MOT_CH,), jnp.int32),
        pltpu.VMEM((_WTAB,), jnp.int32),
        pltpu.SemaphoreType.DMA,
        pltpu.SemaphoreType.DMA,
    ],
)
def _hist_kernel(mg_hbm, mp_hbm, out_hbm, gv0, gv1, pv0, pv1, histp,
                 sem_in0, sem_in1):
    gv = (gv0, gv1)
    pv = (pv0, pv1)
    sem_in = (sem_in0, sem_in1)
    c = lax.axis_index("c")
    s = lax.axis_index("s")
    wid = c * 16 + s
    base = wid * _MOT_PER

    def fire_in(off, b):
        pltpu.async_copy(mg_hbm.at[pl.ds(off, _MOT_CH)], gv[b], sem_in[b])
        pltpu.async_copy(mp_hbm.at[pl.ds(off, _MOT_CH)], pv[b], sem_in[b])

    def wait_in(b):
        pltpu.make_async_copy(mg_hbm.at[pl.ds(0, _MOT_CH)], gv[b],
                              sem_in[b]).wait()
        pltpu.make_async_copy(mp_hbm.at[pl.ds(0, _MOT_CH)], pv[b],
                              sem_in[b]).wait()

    fire_in(base, 0)

    @plsc.parallel_loop(0, _WTAB // 16, unroll=2)
    def zfill(i):
        histp[pl.ds(i * 16, 16)] = jnp.zeros((16,), jnp.int32)

    def proc(b, i):
        g = gv[b][pl.ds(i * 16, 16)]
        p = pv[b][pl.ds(i * 16, 16)]
        k = g * _N_SEG + lax.div(p - 1, 200)
        cnt, last = plsc.scan_count(k)
        plsc.addupdate_scatter(histp, [k], cnt, mask=last)

    def pair_body(t, carry):
        for b in (0, 1):
            j = t * 2 + b

            @pl.when(j + 1 < _MOT_FULL)
            def _():
                fire_in(base + (j + 1) * _MOT_CH, 1 - b)

            wait_in(b)

            def kbody(i, cc):
                proc(b, i)
                return cc

            lax.fori_loop(0, _MOT_CH // 16, kbody, 0)
        return carry

    lax.fori_loop(0, _MOT_FULL // 2, pair_body, 0)

    def do_rem(rem):
        off = base + _MOT_FULL * _MOT_CH
        pltpu.sync_copy(mg_hbm.at[pl.ds(off, rem)], gv0.at[pl.ds(0, rem)])
        pltpu.sync_copy(mp_hbm.at[pl.ds(off, rem)], pv0.at[pl.ds(0, rem)])

        def kbody(i, cc):
            proc(0, i)
            return cc

        lax.fori_loop(0, rem // 16, kbody, 0)

    @pl.when(wid < 31)
    def _():
        do_rem(_MOT_REM0)

    @pl.when(wid == 31)
    def _():
        do_rem(_MOT_REM1)

    pltpu.sync_copy(histp, out_hbm.at[wid])


def _mid_body(w_ref, b_ref, hp_ref, out_ref):
    hist = jnp.sum(hp_ref[...], axis=0).astype(jnp.float32)
    sp = lax.broadcasted_iota(jnp.int32, (_N_SEG, _N_SEG), 0)
    tg = lax.broadcasted_iota(jnp.int32, (_N_SEG, _N_SEG), 1)
    agg = jnp.zeros((_N_SEG, _N_SEG), jnp.float32)
    for i, (r, bw) in enumerate(((10, 2000.0), (5, 1000.0), (2, 400.0))):
        agg = agg + jnp.where((sp // r) == (tg // r), w_ref[i] / bw, 0.0)
    bsum = b_ref[0] + b_ref[1] + b_ref[2]
    v = lax.dot(hist, agg, preferred_element_type=jnp.float32) + bsum
    mx = jnp.max(v, axis=1, keepdims=True)
    z = jnp.sum(jnp.exp(v - mx), axis=1, keepdims=True) * 200.0
    out_ref[...] = v - mx - jnp.log(z)


_mid_call = pl.pallas_call(
    _mid_body,
    out_shape=jax.ShapeDtypeStruct((_N_GENES, _N_SEG), jnp.float32),
    in_specs=[
        pl.BlockSpec(memory_space=pltpu.SMEM),
        pl.BlockSpec(memory_space=pltpu.SMEM),
        pl.BlockSpec(memory_space=pltpu.VMEM),
    ],
)


@functools.partial(
    pl.kernel,
    mesh=_mesh,
    out_type=jax.ShapeDtypeStruct((_N_FRAG,), jnp.float32),
    compiler_params=pltpu.CompilerParams(needs_layout_passes=False),
    scratch_types=[
        pltpu.VMEM((_WTAB,), jnp.float32),
        pltpu.VMEM((_FR_CH,), jnp.int32),
        pltpu.VMEM((_FR_CH,), jnp.int32),
        pltpu.VMEM((_FR_CH,), jnp.int32),
        pltpu.VMEM((_FR_CH,), jnp.int32),
        pltpu.VMEM((_FR_CH,), jnp.float32),
        pltpu.VMEM((_FR_CH,), jnp.float32),
        pltpu.SemaphoreType.DMA,
        pltpu.SemaphoreType.DMA,
        pltpu.SemaphoreType.DMA,
        pltpu.SemaphoreType.DMA,
    ],
)
def _gather_kernel(w_hbm, x_hbm, g_hbm, out_hbm, wtab, xv0, xv1, gv0, gv1,
                   ov0, ov1, sem_in0, sem_in1, sem_out0, sem_out1):
    xv = (xv0, xv1)
    gv = (gv0, gv1)
    ov = (ov0, ov1)
    c = lax.axis_index("c")
    s = lax.axis_index("s")
    wid = c * 16 + s
    sem_in = (sem_in0, sem_in1)
    sem_out = (sem_out0, sem_out1)
    pltpu.sync_copy(w_hbm, wtab)
    base = wid * _FR_PER

    def fire_in(off, b):
        pltpu.async_copy(x_hbm.at[pl.ds(off, _FR_CH)], xv[b], sem_in[b])
        pltpu.async_copy(g_hbm.at[pl.ds(off, _FR_CH)], gv[b], sem_in[b])

    def wait_in(b):
        pltpu.make_async_copy(x_hbm.at[pl.ds(0, _FR_CH)], xv[b],
                              sem_in[b]).wait()
        pltpu.make_async_copy(g_hbm.at[pl.ds(0, _FR_CH)], gv[b],
                              sem_in[b]).wait()

    def gbody(b, i):
        g = gv[b][pl.ds(i * 16, 16)]
        x = xv[b][pl.ds(i * 16, 16)]
        k = g * _N_SEG + lax.div(x, 200)
        ov[b][pl.ds(i * 16, 16)] = plsc.load_gather(wtab, [k])

    fire_in(base, 0)

    def pair_body(t, carry):
        for b in (0, 1):
            j = t * 2 + b

            @pl.when(j + 1 < _FR_FULL)
            def _():
                fire_in(base + (j + 1) * _FR_CH, 1 - b)

            wait_in(b)

            @pl.when(j >= 2)
            def _():
                pltpu.make_async_copy(ov[b], out_hbm.at[pl.ds(0, _FR_CH)],
                                      sem_out[b]).wait()

            def gb(i, cc):
                gbody(b, i)
                return cc

            lax.fori_loop(0, _FR_CH // 16, gb, 0)

            pltpu.async_copy(ov[b], out_hbm.at[pl.ds(base + j * _FR_CH,
                                                     _FR_CH)], sem_out[b])
        return carry

    lax.fori_loop(0, _FR_FULL // 2, pair_body, 0)
    for b in (0, 1):
        pltpu.make_async_copy(ov[b], out_hbm.at[pl.ds(0, _FR_CH)],
                              sem_out[b]).wait()

    def do_rem(rem):
        off = base + _FR_FULL * _FR_CH
        pltpu.sync_copy(x_hbm.at[pl.ds(off, rem)], xv0.at[pl.ds(0, rem)])
        pltpu.sync_copy(g_hbm.at[pl.ds(off, rem)], gv0.at[pl.ds(0, rem)])

        def gb(i, cc):
            gbody(0, i)
            return cc

        lax.fori_loop(0, rem // 16, gb, 0)
        pltpu.sync_copy(ov0.at[pl.ds(0, rem)], out_hbm.at[pl.ds(off, rem)])

    @pl.when(wid < 31)
    def _():
        do_rem(_FR_REM0)

    @pl.when(wid == 31)
    def _():
        do_rem(_FR_REM1)


def kernel(predictor_W, predictor_b, coordinates, frag_local_gene_ix,
           motif_local_gene_ix, motif_positions, genes_oi):
    x = coordinates[:, 0].astype(jnp.int32)
    fg = frag_local_gene_ix.astype(jnp.int32)
    mg = motif_local_gene_ix.astype(jnp.int32)
    mp = motif_positions.astype(jnp.int32)
    parts = _hist_kernel(mg, mp)
    hp = parts.reshape(32, _N_GENES, _N_SEG)
    w2d = _mid_call(predictor_W.astype(jnp.float32),
                    predictor_b.astype(jnp.float32), hp)
    return _gather_kernel(w2d.reshape(-1), x, fg)


# speedup vs baseline: 1.1148x; 1.1148x over previous
import functools

import jax
import jax.numpy as jnp
from jax import lax
from jax.experimental import pallas as pl
from jax.experimental.pallas import tpu as pltpu
from jax.experimental.pallas import tpu_sc as plsc

_N_GENES = 1024
_N_SEG = 100
_WTAB = _N_GENES * _N_SEG
_HSIZE = 102912
_DUMMY = _WTAB
_N_FRAG = 1_000_000
_N_MOT = 2_000_000

_MOT_PER = 62496
_MOT_CH = 2048
_MOT_FULL = 30
_MOT_REM0 = _MOT_PER - _MOT_FULL * _MOT_CH
_MOT_REM1 = (_N_MOT - 31 * _MOT_PER) - _MOT_FULL * _MOT_CH
_MSLICE = _WTAB // 16

_FR_PER = 31248
_FR_CH = 1024
_FR_FULL = 30
_FR_REM0 = _FR_PER - _FR_FULL * _FR_CH
_FR_REM1 = (_N_FRAG - 31 * _FR_PER) - _FR_FULL * _FR_CH

_mesh = plsc.VectorSubcoreMesh(core_axis_name="c", subcore_axis_name="s")


@functools.partial(
    pl.kernel,
    mesh=_mesh,
    out_type=jax.ShapeDtypeStruct((32, _WTAB), jnp.int32),
    compiler_params=pltpu.CompilerParams(needs_layout_passes=False),
    scratch_types=[
        pltpu.VMEM((_MOT_CH,), jnp.int32),
        pltpu.VMEM((_MOT_CH,), jnp.int32),
        pltpu.VMEM((_MOT_CH,), jnp.int32),
        pltpu.VMEM((_MOT_CH,), jnp.int32),
        pltpu.VMEM((_WTAB,), jnp.int32),
        pltpu.SemaphoreType.DMA,
        pltpu.SemaphoreType.DMA,
    ],
)
def _hist_kernel(mg_hbm, mp_hbm, out_hbm, gv0, gv1, pv0, pv1, histp,
                 sem_in0, sem_in1):
    gv = (gv0, gv1)
    pv = (pv0, pv1)
    sem_in = (sem_in0, sem_in1)
    c = lax.axis_index("c")
    s = lax.axis_index("s")
    wid = c * 16 + s
    base = wid * _MOT_PER

    def fire_in(off, b):
        pltpu.async_copy(mg_hbm.at[pl.ds(off, _MOT_CH)], gv[b], sem_in[b])
        pltpu.async_copy(mp_hbm.at[pl.ds(off, _MOT_CH)], pv[b], sem_in[b])

    def wait_in(b):
        pltpu.make_async_copy(mg_hbm.at[pl.ds(0, _MOT_CH)], gv[b],
                              sem_in[b]).wait()
        pltpu.make_async_copy(mp_hbm.at[pl.ds(0, _MOT_CH)], pv[b],
                              sem_in[b]).wait()

    fire_in(base, 0)

    @plsc.parallel_loop(0, _WTAB // 16, unroll=2)
    def zfill(i):
        histp[pl.ds(i * 16, 16)] = jnp.zeros((16,), jnp.int32)

    def proc(b, i):
        g = gv[b][pl.ds(i * 16, 16)]
        p = pv[b][pl.ds(i * 16, 16)]
        k = g * _N_SEG + lax.div(p - 1, 200)
        plsc.addupdate_scatter(histp, [k], jnp.ones((16,), jnp.int32))

    def pair_body(t, carry):
        for b in (0, 1):
            j = t * 2 + b

            @pl.when(j + 1 < _MOT_FULL)
            def _():
                fire_in(base + (j + 1) * _MOT_CH, 1 - b)

            wait_in(b)

            def kbody(i, cc):
                proc(b, i)
                return cc

            lax.fori_loop(0, _MOT_CH // 16, kbody, 0)
        return carry

    lax.fori_loop(0, _MOT_FULL // 2, pair_body, 0)

    def do_rem(rem):
        off = base + _MOT_FULL * _MOT_CH
        pltpu.sync_copy(mg_hbm.at[pl.ds(off, rem)], gv0.at[pl.ds(0, rem)])
        pltpu.sync_copy(mp_hbm.at[pl.ds(off, rem)], pv0.at[pl.ds(0, rem)])

        def kbody(i, cc):
            proc(0, i)
            return cc

        lax.fori_loop(0, rem // 16, kbody, 0)

    @pl.when(wid < 31)
    def _():
        do_rem(_MOT_REM0)

    @pl.when(wid == 31)
    def _():
        do_rem(_MOT_REM1)

    pltpu.sync_copy(histp, out_hbm.at[wid])


def _mid_body(w_ref, b_ref, hp_ref, out_ref):
    hist = jnp.sum(hp_ref[...], axis=0).astype(jnp.float32)
    sp = lax.broadcasted_iota(jnp.int32, (_N_SEG, _N_SEG), 0)
    tg = lax.broadcasted_iota(jnp.int32, (_N_SEG, _N_SEG), 1)
    agg = jnp.zeros((_N_SEG, _N_SEG), jnp.float32)
    for i, (r, bw) in enumerate(((10, 2000.0), (5, 1000.0), (2, 400.0))):
        agg = agg + jnp.where((sp // r) == (tg // r), w_ref[i] / bw, 0.0)
    bsum = b_ref[0] + b_ref[1] + b_ref[2]
    v = lax.dot(hist, agg, preferred_element_type=jnp.float32) + bsum
    mx = jnp.max(v, axis=1, keepdims=True)
    z = jnp.sum(jnp.exp(v - mx), axis=1, keepdims=True) * 200.0
    out_ref[...] = v - mx - jnp.log(z)


_mid_call = pl.pallas_call(
    _mid_body,
    out_shape=jax.ShapeDtypeStruct((_N_GENES, _N_SEG), jnp.float32),
    in_specs=[
        pl.BlockSpec(memory_space=pltpu.SMEM),
        pl.BlockSpec(memory_space=pltpu.SMEM),
        pl.BlockSpec(memory_space=pltpu.VMEM),
    ],
)


@functools.partial(
    pl.kernel,
    mesh=_mesh,
    out_type=jax.ShapeDtypeStruct((_N_FRAG,), jnp.float32),
    compiler_params=pltpu.CompilerParams(needs_layout_passes=False),
    scratch_types=[
        pltpu.VMEM((_WTAB,), jnp.float32),
        pltpu.VMEM((_FR_CH,), jnp.int32),
        pltpu.VMEM((_FR_CH,), jnp.int32),
        pltpu.VMEM((_FR_CH,), jnp.int32),
        pltpu.VMEM((_FR_CH,), jnp.int32),
        pltpu.VMEM((_FR_CH,), jnp.float32),
        pltpu.VMEM((_FR_CH,), jnp.float32),
        pltpu.SemaphoreType.DMA,
        pltpu.SemaphoreType.DMA,
        pltpu.SemaphoreType.DMA,
        pltpu.SemaphoreType.DMA,
    ],
)
def _gather_kernel(w_hbm, x_hbm, g_hbm, out_hbm, wtab, xv0, xv1, gv0, gv1,
                   ov0, ov1, sem_in0, sem_in1, sem_out0, sem_out1):
    xv = (xv0, xv1)
    gv = (gv0, gv1)
    ov = (ov0, ov1)
    c = lax.axis_index("c")
    s = lax.axis_index("s")
    wid = c * 16 + s
    sem_in = (sem_in0, sem_in1)
    sem_out = (sem_out0, sem_out1)
    pltpu.sync_copy(w_hbm, wtab)
    base = wid * _FR_PER

    def fire_in(off, b):
        pltpu.async_copy(x_hbm.at[pl.ds(off, _FR_CH)], xv[b], sem_in[b])
        pltpu.async_copy(g_hbm.at[pl.ds(off, _FR_CH)], gv[b], sem_in[b])

    def wait_in(b):
        pltpu.make_async_copy(x_hbm.at[pl.ds(0, _FR_CH)], xv[b],
                              sem_in[b]).wait()
        pltpu.make_async_copy(g_hbm.at[pl.ds(0, _FR_CH)], gv[b],
                              sem_in[b]).wait()

    def gbody(b, i):
        g = gv[b][pl.ds(i * 16, 16)]
        x = xv[b][pl.ds(i * 16, 16)]
        k = g * _N_SEG + lax.div(x, 200)
        ov[b][pl.ds(i * 16, 16)] = plsc.load_gather(wtab, [k])

    fire_in(base, 0)

    def pair_body(t, carry):
        for b in (0, 1):
            j = t * 2 + b

            @pl.when(j + 1 < _FR_FULL)
            def _():
                fire_in(base + (j + 1) * _FR_CH, 1 - b)

            wait_in(b)

            @pl.when(j >= 2)
            def _():
                pltpu.make_async_copy(ov[b], out_hbm.at[pl.ds(0, _FR_CH)],
                                      sem_out[b]).wait()

            def gb(i, cc):
                gbody(b, i)
                return cc

            lax.fori_loop(0, _FR_CH // 16, gb, 0)

            pltpu.async_copy(ov[b], out_hbm.at[pl.ds(base + j * _FR_CH,
                                                     _FR_CH)], sem_out[b])
        return carry

    lax.fori_loop(0, _FR_FULL // 2, pair_body, 0)
    for b in (0, 1):
        pltpu.make_async_copy(ov[b], out_hbm.at[pl.ds(0, _FR_CH)],
                              sem_out[b]).wait()

    def do_rem(rem):
        off = base + _FR_FULL * _FR_CH
        pltpu.sync_copy(x_hbm.at[pl.ds(off, rem)], xv0.at[pl.ds(0, rem)])
        pltpu.sync_copy(g_hbm.at[pl.ds(off, rem)], gv0.at[pl.ds(0, rem)])

        def gb(i, cc):
            gbody(0, i)
            return cc

        lax.fori_loop(0, rem // 16, gb, 0)
        pltpu.sync_copy(ov0.at[pl.ds(0, rem)], out_hbm.at[pl.ds(off, rem)])

    @pl.when(wid < 31)
    def _():
        do_rem(_FR_REM0)

    @pl.when(wid == 31)
    def _():
        do_rem(_FR_REM1)


def kernel(predictor_W, predictor_b, coordinates, frag_local_gene_ix,
           motif_local_gene_ix, motif_positions, genes_oi):
    x = coordinates[:, 0].astype(jnp.int32)
    fg = frag_local_gene_ix.astype(jnp.int32)
    mg = motif_local_gene_ix.astype(jnp.int32)
    mp = motif_positions.astype(jnp.int32)
    parts = _hist_kernel(mg, mp)
    hp = parts.reshape(32, _N_GENES, _N_SEG)
    w2d = _mid_call(predictor_W.astype(jnp.float32),
                    predictor_b.astype(jnp.float32), hp)
    return _gather_kernel(w2d.reshape(-1), x, fg)


# speedup vs baseline: 1.6148x; 1.4485x over previous
import functools

import jax
import jax.numpy as jnp
from jax import lax
from jax.experimental import pallas as pl
from jax.experimental.pallas import tpu as pltpu
from jax.experimental.pallas import tpu_sc as plsc

_N_GENES = 1024
_N_SEG = 100
_WTAB = _N_GENES * _N_SEG
_HSIZE = 102912
_DUMMY = _WTAB
_N_FRAG = 1_000_000
_N_MOT = 2_000_000

_MOT_PER = 62496
_MOT_CH = 2048
_MOT_FULL = 30
_MOT_REM0 = _MOT_PER - _MOT_FULL * _MOT_CH
_MOT_REM1 = (_N_MOT - 31 * _MOT_PER) - _MOT_FULL * _MOT_CH

_FR_PER = 31248
_FR_CH = 1024
_FR_FULL = 30
_FR_REM0 = _FR_PER - _FR_FULL * _FR_CH
_FR_REM1 = (_N_FRAG - 31 * _FR_PER) - _FR_FULL * _FR_CH

_mesh = plsc.VectorSubcoreMesh(core_axis_name="c", subcore_axis_name="s")


@functools.partial(
    pl.kernel,
    mesh=_mesh,
    out_type=jax.ShapeDtypeStruct((2, _WTAB), jnp.float32),
    scratch_types=[
        pltpu.VMEM((_MOT_CH,), jnp.int32),
        pltpu.VMEM((_MOT_CH,), jnp.int32),
        pltpu.VMEM((_MOT_CH,), jnp.int32),
        pltpu.VMEM((_MOT_CH,), jnp.int32),
        pltpu.VMEM((16, 128), jnp.int32),
        pltpu.VMEM((16, 128), jnp.int32),
        pltpu.VMEM((128,), jnp.float32),
        pltpu.VMEM((_HSIZE // 16,), jnp.float32),
        pltpu.VMEM_SHARED((_HSIZE,), jnp.float32),
        pltpu.SemaphoreType.DMA,
        pltpu.SemaphoreType.DMA,
        pltpu.SemaphoreType.DMA,
        pltpu.SemaphoreType.DMA,
    ],
)
def _hist_kernel(mg_hbm, mp_hbm, out_hbm, gv0, gv1, pv0, pv1, keys0, keys1,
                 ones_v, zbuf, hist_sh, sem_in0, sem_in1, sem_sc0, sem_sc1):
    gv = (gv0, gv1)
    pv = (pv0, pv1)
    keys = (keys0, keys1)
    c = lax.axis_index("c")
    s = lax.axis_index("s")
    wid = c * 16 + s
    zslab = _HSIZE // 16
    sem_in = (sem_in0, sem_in1)
    sem_sc = (sem_sc0, sem_sc1)

    def zfill(i, carry):
        zbuf[pl.ds(i * 16, 16)] = jnp.zeros((16,), jnp.float32)
        return carry

    lax.fori_loop(0, zslab // 16, zfill, 0)
    for i in range(8):
        ones_v[pl.ds(i * 16, 16)] = jnp.ones((16,), jnp.float32)
    pltpu.sync_copy(zbuf, hist_sh.at[pl.ds(s * zslab, zslab)])
    plsc.subcore_barrier()

    base = wid * _MOT_PER

    def fire_in(off, b):
        pltpu.async_copy(mg_hbm.at[pl.ds(off, _MOT_CH)], gv[b], sem_in[b])
        pltpu.async_copy(mp_hbm.at[pl.ds(off, _MOT_CH)], pv[b], sem_in[b])

    def wait_in(b):
        pltpu.make_async_copy(mg_hbm.at[pl.ds(0, _MOT_CH)], gv[b],
                              sem_in[b]).wait()
        pltpu.make_async_copy(mp_hbm.at[pl.ds(0, _MOT_CH)], pv[b],
                              sem_in[b]).wait()

    def compute_keys(b, i):
        g = gv[b][pl.ds(i * 16, 16)]
        p = pv[b][pl.ds(i * 16, 16)]
        k = g * _N_SEG + lax.div(p - 1, 200)
        keys[b][lax.div(i, 8), pl.ds(lax.rem(i, 8) * 16, 16)] = k

    def fire_scatter(b, nrows=16):
        for m in range(nrows):
            pltpu.async_copy(ones_v, hist_sh.at[keys[b].at[m]], sem_sc[b],
                             add=True)

    def drain_scatter(b, nrows=16):
        for m in range(nrows):
            pltpu.make_async_copy(ones_v, hist_sh.at[keys[b].at[m]],
                                  sem_sc[b]).wait()

    fire_in(base, 0)

    def pair_body(t, carry):
        for b in (0, 1):
            j = t * 2 + b

            @pl.when(j + 1 < _MOT_FULL)
            def _():
                fire_in(base + (j + 1) * _MOT_CH, 1 - b)

            wait_in(b)

            @pl.when(j >= 2)
            def _():
                drain_scatter(b)

            @plsc.parallel_loop(0, _MOT_CH // 16, unroll=2)
            def kbody(i):
                compute_keys(b, i)

            fire_scatter(b)
        return carry

    lax.fori_loop(0, _MOT_FULL // 2, pair_body, 0)
    drain_scatter(0)
    drain_scatter(1)

    def do_rem(rem):
        off = base + _MOT_FULL * _MOT_CH
        pltpu.sync_copy(mg_hbm.at[pl.ds(off, rem)], gv0.at[pl.ds(0, rem)])
        pltpu.sync_copy(mp_hbm.at[pl.ds(off, rem)], pv0.at[pl.ds(0, rem)])
        nv = rem // 16
        nrows = (nv + 7) // 8

        @plsc.parallel_loop(0, nv, unroll=2)
        def kbody(i):
            compute_keys(0, i)

        def dbody(i, cc):
            keys0[lax.div(i, 8), pl.ds(lax.rem(i, 8) * 16, 16)] = jnp.full(
                (16,), _DUMMY, jnp.int32)
            return cc

        lax.fori_loop(nv, nrows * 8, dbody, 0)
        fire_scatter(0, nrows)
        drain_scatter(0, nrows)

    @pl.when(wid < 31)
    def _():
        do_rem(_MOT_REM0)

    @pl.when(wid == 31)
    def _():
        do_rem(_MOT_REM1)

    plsc.subcore_barrier()

    @pl.when(s == 0)
    def _():
        pltpu.sync_copy(hist_sh.at[pl.ds(0, _WTAB)], out_hbm.at[c])


def _mid_body(w_ref, b_ref, hp_ref, out_ref):
    hist = hp_ref[0] + hp_ref[1]
    sp = lax.broadcasted_iota(jnp.int32, (_N_SEG, _N_SEG), 0)
    tg = lax.broadcasted_iota(jnp.int32, (_N_SEG, _N_SEG), 1)
    agg = jnp.zeros((_N_SEG, _N_SEG), jnp.float32)
    for i, (r, bw) in enumerate(((10, 2000.0), (5, 1000.0), (2, 400.0))):
        agg = agg + jnp.where((sp // r) == (tg // r), w_ref[i] / bw, 0.0)
    bsum = b_ref[0] + b_ref[1] + b_ref[2]
    v = lax.dot(hist, agg, preferred_element_type=jnp.float32) + bsum
    mx = jnp.max(v, axis=1, keepdims=True)
    z = jnp.sum(jnp.exp(v - mx), axis=1, keepdims=True) * 200.0
    out_ref[...] = v - mx - jnp.log(z)


_mid_call = pl.pallas_call(
    _mid_body,
    out_shape=jax.ShapeDtypeStruct((_N_GENES, _N_SEG), jnp.float32),
    in_specs=[
        pl.BlockSpec(memory_space=pltpu.SMEM),
        pl.BlockSpec(memory_space=pltpu.SMEM),
        pl.BlockSpec(memory_space=pltpu.VMEM),
    ],
)


@functools.partial(
    pl.kernel,
    mesh=_mesh,
    out_type=jax.ShapeDtypeStruct((_N_FRAG,), jnp.float32),
    compiler_params=pltpu.CompilerParams(needs_layout_passes=False),
    scratch_types=[
        pltpu.VMEM((_WTAB,), jnp.float32),
        pltpu.VMEM((_FR_CH,), jnp.int32),
        pltpu.VMEM((_FR_CH,), jnp.int32),
        pltpu.VMEM((_FR_CH,), jnp.int32),
        pltpu.VMEM((_FR_CH,), jnp.int32),
        pltpu.VMEM((_FR_CH,), jnp.float32),
        pltpu.VMEM((_FR_CH,), jnp.float32),
        pltpu.SemaphoreType.DMA,
        pltpu.SemaphoreType.DMA,
        pltpu.SemaphoreType.DMA,
        pltpu.SemaphoreType.DMA,
    ],
)
def _gather_kernel(w_hbm, x_hbm, g_hbm, out_hbm, wtab, xv0, xv1, gv0, gv1,
                   ov0, ov1, sem_in0, sem_in1, sem_out0, sem_out1):
    xv = (xv0, xv1)
    gv = (gv0, gv1)
    ov = (ov0, ov1)
    c = lax.axis_index("c")
    s = lax.axis_index("s")
    wid = c * 16 + s
    sem_in = (sem_in0, sem_in1)
    sem_out = (sem_out0, sem_out1)
    pltpu.sync_copy(w_hbm, wtab)
    base = wid * _FR_PER

    def fire_in(off, b):
        pltpu.async_copy(x_hbm.at[pl.ds(off, _FR_CH)], xv[b], sem_in[b])
        pltpu.async_copy(g_hbm.at[pl.ds(off, _FR_CH)], gv[b], sem_in[b])

    def wait_in(b):
        pltpu.make_async_copy(x_hbm.at[pl.ds(0, _FR_CH)], xv[b],
                              sem_in[b]).wait()
        pltpu.make_async_copy(g_hbm.at[pl.ds(0, _FR_CH)], gv[b],
                              sem_in[b]).wait()

    def gbody(b, i):
        g = gv[b][pl.ds(i * 16, 16)]
        x = xv[b][pl.ds(i * 16, 16)]
        k = g * _N_SEG + lax.div(x, 200)
        ov[b][pl.ds(i * 16, 16)] = plsc.load_gather(wtab, [k])

    fire_in(base, 0)

    def pair_body(t, carry):
        for b in (0, 1):
            j = t * 2 + b

            @pl.when(j + 1 < _FR_FULL)
            def _():
                fire_in(base + (j + 1) * _FR_CH, 1 - b)

            wait_in(b)

            @pl.when(j >= 2)
            def _():
                pltpu.make_async_copy(ov[b], out_hbm.at[pl.ds(0, _FR_CH)],
                                      sem_out[b]).wait()

            def gb(i, cc):
                gbody(b, i)
                return cc

            lax.fori_loop(0, _FR_CH // 16, gb, 0)
            pltpu.async_copy(ov[b], out_hbm.at[pl.ds(base + j * _FR_CH,
                                                     _FR_CH)], sem_out[b])
        return carry

    lax.fori_loop(0, _FR_FULL // 2, pair_body, 0)
    for b in (0, 1):
        pltpu.make_async_copy(ov[b], out_hbm.at[pl.ds(0, _FR_CH)],
                              sem_out[b]).wait()

    def do_rem(rem):
        off = base + _FR_FULL * _FR_CH
        pltpu.sync_copy(x_hbm.at[pl.ds(off, rem)], xv0.at[pl.ds(0, rem)])
        pltpu.sync_copy(g_hbm.at[pl.ds(off, rem)], gv0.at[pl.ds(0, rem)])

        def gb(i, cc):
            gbody(0, i)
            return cc

        lax.fori_loop(0, rem // 16, gb, 0)
        pltpu.sync_copy(ov0.at[pl.ds(0, rem)], out_hbm.at[pl.ds(off, rem)])

    @pl.when(wid < 31)
    def _():
        do_rem(_FR_REM0)

    @pl.when(wid == 31)
    def _():
        do_rem(_FR_REM1)


def kernel(predictor_W, predictor_b, coordinates, frag_local_gene_ix,
           motif_local_gene_ix, motif_positions, genes_oi):
    x = coordinates[:, 0].astype(jnp.int32)
    fg = frag_local_gene_ix.astype(jnp.int32)
    mg = motif_local_gene_ix.astype(jnp.int32)
    mp = motif_positions.astype(jnp.int32)
    parts = _hist_kernel(mg, mp)
    hp = parts.reshape(2, _N_GENES, _N_SEG)
    w2d = _mid_call(predictor_W.astype(jnp.float32),
                    predictor_b.astype(jnp.float32), hp)
    return _gather_kernel(w2d.reshape(-1), x, fg)
